# Initial kernel scaffold; baseline (speedup 1.0000x reference)
import jax
import jax.numpy as jnp
from jax.experimental import pallas as pl


def kernel(x, edge_index, edge_attr, batch, W1, b1, W2, b2, W3, b3, W_out, b_out):
    raise NotImplementedError("write your pallas kernel here")


# speedup vs baseline: 6.3242x; 6.3242x over previous
import functools

import jax
import jax.numpy as jnp
from jax import lax
from jax.experimental import pallas as pl
from jax.experimental.pallas import tpu as pltpu
from jax.experimental.pallas import tpu_sc as plsc

N_PAD = 10240
ROWS_PS = N_PAD // 16
EB = 128
NB = 80
NBD = 40
E_PAD = 16 * NB * EB
BM = 512


def _mesh():
    return plsc.VectorSubcoreMesh(core_axis_name="c", subcore_axis_name="s")


@functools.partial(
    pl.kernel,
    out_type=jax.ShapeDtypeStruct((2, N_PAD, 128), jnp.float32),
    mesh=_mesh(),
    scratch_types=[
        pltpu.VMEM_SHARED((N_PAD, 128), jnp.float32),
        pltpu.VMEM((NBD, EB), jnp.int32),
        pltpu.VMEM((EB, 128), jnp.float32),
    ],
)
def _deg_kernel(dst32, ones_hbm, zeros_hbm, degp, acc, idx_v, ones_v):
    core = lax.axis_index("c")
    sub = lax.axis_index("s")
    tile = sub * 2 + core
    r0 = sub * ROWS_PS
    pltpu.sync_copy(zeros_hbm.at[pl.ds(r0, ROWS_PS)], acc.at[pl.ds(r0, ROWS_PS)])
    pltpu.sync_copy(ones_hbm, ones_v)
    pltpu.sync_copy(dst32.at[tile], idx_v)
    plsc.subcore_barrier()

    def body(k, carry):
        pltpu.sync_copy(ones_v, acc.at[idx_v.at[k]], add=True)
        return carry

    lax.fori_loop(0, NBD, body, 0)
    plsc.subcore_barrier()
    pltpu.sync_copy(acc.at[pl.ds(r0, ROWS_PS)], degp.at[core].at[pl.ds(r0, ROWS_PS)])


def _make_seg_kernel(C):
    @functools.partial(
        pl.kernel,
        out_type=jax.ShapeDtypeStruct((C, N_PAD, 128), jnp.float32),
        mesh=_mesh(),
        scratch_types=[
            pltpu.VMEM_SHARED((N_PAD, 128), jnp.float32),
            pltpu.VMEM((NB, EB), jnp.int32),
            pltpu.VMEM((NB, EB), jnp.int32),
            pltpu.VMEM((EB, 128), jnp.float32),
            pltpu.SemaphoreType.DMA,
        ],
    )
    def seg(g, src16, dst16, out, acc, src_v, dst_v, stage, sem):
        core = lax.axis_index("c")
        sub = lax.axis_index("s")
        r0 = sub * ROWS_PS
        pltpu.sync_copy(src16.at[sub], src_v)
        pltpu.sync_copy(dst16.at[sub], dst_v)
        for c in range(C):

            @pl.when(core == (c % 2))
            def _():
                pltpu.sync_copy(
                    g.at[c].at[pl.ds(r0, ROWS_PS)], acc.at[pl.ds(r0, ROWS_PS)]
                )
                plsc.subcore_barrier()

                def body(k, carry):
                    cp = pltpu.async_copy(g.at[c].at[src_v.at[k]], stage, sem)
                    cp.wait()
                    pltpu.sync_copy(stage, acc.at[dst_v.at[k]], add=True)
                    return carry

                lax.fori_loop(0, NB, body, 0)
                plsc.subcore_barrier()
                pltpu.sync_copy(
                    acc.at[pl.ds(r0, ROWS_PS)], out.at[c].at[pl.ds(r0, ROWS_PS)]
                )

    return seg


_seg2 = _make_seg_kernel(2)
_seg4 = _make_seg_kernel(4)


def _dinv_block(degp_blk):
    deg = degp_blk.sum(axis=0).sum(axis=1, keepdims=True) * (1.0 / 128.0) + 1.0
    return lax.rsqrt(deg)


def _k1_body(degp_ref, x_ref, g0_ref):
    dinv = _dinv_block(degp_ref[...])
    g = x_ref[...] * dinv
    g0_ref[0] = g[:, :128]
    g0_ref[1] = g[:, 128:]


def _apply_k1(degp, x_pad):
    return pl.pallas_call(
        _k1_body,
        grid=(N_PAD // BM,),
        in_specs=[
            pl.BlockSpec((2, BM, 128), lambda m: (0, m, 0)),
            pl.BlockSpec((BM, 256), lambda m: (m, 0)),
        ],
        out_specs=pl.BlockSpec((2, BM, 128), lambda m: (0, m, 0)),
        out_shape=jax.ShapeDtypeStruct((2, N_PAD, 128), jnp.float32),
    )(degp, x_pad)


def _k2_body(degp_ref, p_ref, w_ref, b_ref, g_ref, *, C_in):
    dinv = _dinv_block(degp_ref[...])
    acc = jnp.zeros((BM, 512), jnp.float32)
    for c in range(C_in):
        a = p_ref[c] * dinv
        acc = acc + jnp.dot(
            a, w_ref[c * 128 : (c + 1) * 128, :], preferred_element_type=jnp.float32
        )
    h = jax.nn.relu(acc + b_ref[...])
    g = h * dinv
    for c in range(4):
        g_ref[c] = g[:, c * 128 : (c + 1) * 128]


def _apply_k2(degp, P, W, b):
    C_in = P.shape[0]
    return pl.pallas_call(
        functools.partial(_k2_body, C_in=C_in),
        grid=(N_PAD // BM,),
        in_specs=[
            pl.BlockSpec((2, BM, 128), lambda m: (0, m, 0)),
            pl.BlockSpec((C_in, BM, 128), lambda m: (0, m, 0)),
            pl.BlockSpec((128 * C_in, 512), lambda m: (0, 0)),
            pl.BlockSpec((1, 512), lambda m: (0, 0)),
        ],
        out_specs=pl.BlockSpec((4, BM, 128), lambda m: (0, m, 0)),
        out_shape=jax.ShapeDtypeStruct((4, N_PAD, 128), jnp.float32),
    )(degp, P, W, b)


def _k3_body(degp_ref, p_ref, w3_ref, b3_ref, wo_ref, bo_ref, prob_ref, h_ref):
    dinv = _dinv_block(degp_ref[...])
    acc = jnp.zeros((BM, 512), jnp.float32)
    for c in range(4):
        a = p_ref[c] * dinv
        acc = acc + jnp.dot(
            a, w3_ref[c * 128 : (c + 1) * 128, :], preferred_element_type=jnp.float32
        )
    h = acc + b3_ref[...]
    h_ref[...] = h
    o = jnp.dot(h, wo_ref[...], preferred_element_type=jnp.float32) + bo_ref[...]
    m = jnp.max(o, axis=1, keepdims=True)
    e = jnp.exp(o - m)
    s = jnp.sum(e, axis=1, keepdims=True)
    prob_ref[...] = jnp.clip(e / s, 1e-08, 1.0)


def _apply_k3(degp, P, W3, b3, W_out, b_out):
    return pl.pallas_call(
        _k3_body,
        grid=(N_PAD // BM,),
        in_specs=[
            pl.BlockSpec((2, BM, 128), lambda m: (0, m, 0)),
            pl.BlockSpec((4, BM, 128), lambda m: (0, m, 0)),
            pl.BlockSpec((512, 512), lambda m: (0, 0)),
            pl.BlockSpec((1, 512), lambda m: (0, 0)),
            pl.BlockSpec((512, 128), lambda m: (0, 0)),
            pl.BlockSpec((1, 128), lambda m: (0, 0)),
        ],
        out_specs=[
            pl.BlockSpec((BM, 128), lambda m: (m, 0)),
            pl.BlockSpec((BM, 512), lambda m: (m, 0)),
        ],
        out_shape=[
            jax.ShapeDtypeStruct((N_PAD, 128), jnp.float32),
            jax.ShapeDtypeStruct((N_PAD, 512), jnp.float32),
        ],
    )(degp, P, W3, b3, W_out, b_out)


def kernel(x, edge_index, edge_attr, batch, W1, b1, W2, b2, W3, b3, W_out, b_out):
    n = x.shape[0]
    e = edge_index.shape[1]
    f32 = jnp.float32

    x_pad = jnp.pad(x, ((0, N_PAD - n), (0, 0)))
    fill = jnp.full((E_PAD - e,), N_PAD - 1, jnp.int32)
    srcp = jnp.concatenate([edge_index[0], fill])
    dstp = jnp.concatenate([edge_index[1], fill])
    src16 = srcp.reshape(16, NB, EB)
    dst16 = dstp.reshape(16, NB, EB)
    dst32 = dstp.reshape(32, NBD, EB)
    ones16 = jnp.ones((EB, 128), f32)
    zeros16 = jnp.zeros((N_PAD, 128), f32)

    degp = _deg_kernel(dst32, ones16, zeros16)
    g0 = _apply_k1(degp, x_pad)
    P1 = _seg2(g0, src16, dst16)
    g1 = _apply_k2(degp, P1, W1, b1.reshape(1, -1))
    P2 = _seg4(g1, src16, dst16)
    g2 = _apply_k2(degp, P2, W2, b2.reshape(1, -1))
    P3 = _seg4(g2, src16, dst16)
    prob, h = _apply_k3(degp, P3, W3, b3.reshape(1, -1), W_out, b_out.reshape(1, -1))
    return (prob[:n], h[:n])


# speedup vs baseline: 7.1551x; 1.1314x over previous
import functools

import jax
import jax.numpy as jnp
from jax import lax
from jax.experimental import pallas as pl
from jax.experimental.pallas import tpu as pltpu
from jax.experimental.pallas import tpu_sc as plsc

N_PAD = 10240
ROWS_PS = N_PAD // 16
EB = 128
NB = 80
NBD = 40
NI = 4
E_PAD = 16 * NB * EB
NBUF = 2
BM = 512


def _mesh():
    return plsc.VectorSubcoreMesh(core_axis_name="c", subcore_axis_name="s")


@functools.partial(
    pl.kernel,
    out_type=jax.ShapeDtypeStruct((2, N_PAD, 128), jnp.float32),
    mesh=_mesh(),
    scratch_types=[
        pltpu.VMEM_SHARED((N_PAD, 128), jnp.float32),
        pltpu.VMEM((NBD, EB), jnp.int32),
        pltpu.VMEM((EB, 128), jnp.float32),
    ],
)
def _deg_kernel(dst32, ones_hbm, zeros_hbm, degp, acc, idx_v, ones_v):
    core = lax.axis_index("c")
    sub = lax.axis_index("s")
    tile = sub * 2 + core
    r0 = sub * ROWS_PS
    pltpu.sync_copy(zeros_hbm.at[pl.ds(r0, ROWS_PS)], acc.at[pl.ds(r0, ROWS_PS)])
    pltpu.sync_copy(ones_hbm, ones_v)
    pltpu.sync_copy(dst32.at[tile], idx_v)
    plsc.subcore_barrier()

    def body(k, carry):
        pltpu.sync_copy(ones_v, acc.at[idx_v.at[k]], add=True)
        return carry

    lax.fori_loop(0, NBD, body, 0)
    plsc.subcore_barrier()
    pltpu.sync_copy(acc.at[pl.ds(r0, ROWS_PS)], degp.at[core].at[pl.ds(r0, ROWS_PS)])


def _make_seg_kernel(C):
    @functools.partial(
        pl.kernel,
        out_type=jax.ShapeDtypeStruct((C, N_PAD, 128), jnp.float32),
        mesh=_mesh(),
        scratch_types=[
            pltpu.VMEM_SHARED((N_PAD, 128), jnp.float32),
            pltpu.VMEM((NI, 2, EB), jnp.int32),
            pltpu.VMEM((NBUF, EB, 128), jnp.float32),
            [pltpu.SemaphoreType.DMA] * NI,
            [pltpu.SemaphoreType.DMA] * NBUF,
            [pltpu.SemaphoreType.DMA] * NBUF,
        ],
    )
    def seg(g, edges, out, acc, idx_v, stage, isems, gsems, ssems):
        core = lax.axis_index("c")
        sub = lax.axis_index("s")
        r0 = sub * ROWS_PS
        for c in range(C):

            @pl.when(core == (c % 2))
            def _():
                pltpu.sync_copy(
                    g.at[c].at[pl.ds(r0, ROWS_PS)], acc.at[pl.ds(r0, ROWS_PS)]
                )
                plsc.subcore_barrier()

                def idx_load(k, j, start):
                    mk = pltpu.async_copy if start else pltpu.make_async_copy
                    return mk(edges.at[sub, k], idx_v.at[j], isems[j])

                def gather(k, j, b, start):
                    mk = pltpu.async_copy if start else pltpu.make_async_copy
                    return mk(g.at[c].at[idx_v.at[j, 0]], stage.at[b], gsems[b])

                def scatter(k, j, b, start):
                    if start:
                        return pltpu.async_copy(
                            stage.at[b], acc.at[idx_v.at[j, 1]], ssems[b], add=True
                        )
                    return pltpu.make_async_copy(
                        stage.at[b], acc.at[idx_v.at[j, 1]], ssems[b]
                    )

                for j in range(3):
                    idx_load(j, j, True)
                idx_load(0, 0, False).wait()
                gather(0, 0, 0, True)

                def step(k, u, do_g, do_i):
                    if do_g:
                        jn = (u + 1) % NI
                        idx_load(k + 1, jn, False).wait()
                        gather(k + 1, jn, (u + 1) % NBUF, True)
                    j = u % NI
                    b = u % NBUF
                    gather(k, j, b, False).wait()
                    scatter(k, j, b, True)
                    scatter(k, j, b, False).wait()
                    if do_i:
                        idx_load(k + 3, (u + 3) % NI, True)

                def body(k4, carry):
                    k0 = k4 * NI
                    for u in range(NI):
                        step(k0 + u, u, True, True)
                    return carry

                lax.fori_loop(0, NB // NI - 1, body, 0)
                for u in range(NI):
                    k = NB - NI + u
                    step(k, u, k + 1 < NB, k + 3 < NB)
                plsc.subcore_barrier()
                pltpu.sync_copy(
                    acc.at[pl.ds(r0, ROWS_PS)], out.at[c].at[pl.ds(r0, ROWS_PS)]
                )

    return seg


_seg2 = _make_seg_kernel(2)
_seg4 = _make_seg_kernel(4)


def _dinv_block(degp_blk):
    deg = degp_blk.sum(axis=0).sum(axis=1, keepdims=True) * (1.0 / 128.0) + 1.0
    return lax.rsqrt(deg)


def _k1_body(degp_ref, x_ref, g0_ref):
    dinv = _dinv_block(degp_ref[...])
    g = x_ref[...] * dinv
    g0_ref[0] = g[:, :128]
    g0_ref[1] = g[:, 128:]


def _apply_k1(degp, x_pad):
    return pl.pallas_call(
        _k1_body,
        grid=(N_PAD // BM,),
        in_specs=[
            pl.BlockSpec((2, BM, 128), lambda m: (0, m, 0)),
            pl.BlockSpec((BM, 256), lambda m: (m, 0)),
        ],
        out_specs=pl.BlockSpec((2, BM, 128), lambda m: (0, m, 0)),
        out_shape=jax.ShapeDtypeStruct((2, N_PAD, 128), jnp.float32),
    )(degp, x_pad)


def _k2_body(degp_ref, p_ref, w_ref, b_ref, g_ref, *, C_in):
    dinv = _dinv_block(degp_ref[...])
    acc = jnp.zeros((BM, 512), jnp.float32)
    for c in range(C_in):
        a = p_ref[c] * dinv
        acc = acc + jnp.dot(
            a, w_ref[c * 128 : (c + 1) * 128, :], preferred_element_type=jnp.float32
        )
    h = jax.nn.relu(acc + b_ref[...])
    g = h * dinv
    for c in range(4):
        g_ref[c] = g[:, c * 128 : (c + 1) * 128]


def _apply_k2(degp, P, W, b):
    C_in = P.shape[0]
    return pl.pallas_call(
        functools.partial(_k2_body, C_in=C_in),
        grid=(N_PAD // BM,),
        in_specs=[
            pl.BlockSpec((2, BM, 128), lambda m: (0, m, 0)),
            pl.BlockSpec((C_in, BM, 128), lambda m: (0, m, 0)),
            pl.BlockSpec((128 * C_in, 512), lambda m: (0, 0)),
            pl.BlockSpec((1, 512), lambda m: (0, 0)),
        ],
        out_specs=pl.BlockSpec((4, BM, 128), lambda m: (0, m, 0)),
        out_shape=jax.ShapeDtypeStruct((4, N_PAD, 128), jnp.float32),
    )(degp, P, W, b)


def _k3_body(degp_ref, p_ref, w3_ref, b3_ref, wo_ref, bo_ref, prob_ref, h_ref):
    dinv = _dinv_block(degp_ref[...])
    acc = jnp.zeros((BM, 512), jnp.float32)
    for c in range(4):
        a = p_ref[c] * dinv
        acc = acc + jnp.dot(
            a, w3_ref[c * 128 : (c + 1) * 128, :], preferred_element_type=jnp.float32
        )
    h = acc + b3_ref[...]
    h_ref[...] = h
    o = jnp.dot(h, wo_ref[...], preferred_element_type=jnp.float32) + bo_ref[...]
    m = jnp.max(o, axis=1, keepdims=True)
    e = jnp.exp(o - m)
    s = jnp.sum(e, axis=1, keepdims=True)
    prob_ref[...] = jnp.clip(e / s, 1e-08, 1.0)


def _apply_k3(degp, P, W3, b3, W_out, b_out):
    return pl.pallas_call(
        _k3_body,
        grid=(N_PAD // BM,),
        in_specs=[
            pl.BlockSpec((2, BM, 128), lambda m: (0, m, 0)),
            pl.BlockSpec((4, BM, 128), lambda m: (0, m, 0)),
            pl.BlockSpec((512, 512), lambda m: (0, 0)),
            pl.BlockSpec((1, 512), lambda m: (0, 0)),
            pl.BlockSpec((512, 128), lambda m: (0, 0)),
            pl.BlockSpec((1, 128), lambda m: (0, 0)),
        ],
        out_specs=[
            pl.BlockSpec((BM, 128), lambda m: (m, 0)),
            pl.BlockSpec((BM, 512), lambda m: (m, 0)),
        ],
        out_shape=[
            jax.ShapeDtypeStruct((N_PAD, 128), jnp.float32),
            jax.ShapeDtypeStruct((N_PAD, 512), jnp.float32),
        ],
    )(degp, P, W3, b3, W_out, b_out)


def kernel(x, edge_index, edge_attr, batch, W1, b1, W2, b2, W3, b3, W_out, b_out):
    n = x.shape[0]
    e = edge_index.shape[1]
    f32 = jnp.float32

    x_pad = jnp.pad(x, ((0, N_PAD - n), (0, 0)))
    fill = jnp.full((E_PAD - e,), N_PAD - 1, jnp.int32)
    srcp = jnp.concatenate([edge_index[0], fill])
    dstp = jnp.concatenate([edge_index[1], fill])
    edges16 = jnp.stack([srcp.reshape(16, NB, EB), dstp.reshape(16, NB, EB)], axis=2)
    dst32 = dstp.reshape(32, NBD, EB)
    ones16 = jnp.ones((EB, 128), f32)
    zeros16 = jnp.zeros((N_PAD, 128), f32)

    degp = _deg_kernel(dst32, ones16, zeros16)
    g0 = _apply_k1(degp, x_pad)
    P1 = _seg2(g0, edges16)
    g1 = _apply_k2(degp, P1, W1, b1.reshape(1, -1))
    P2 = _seg4(g1, edges16)
    g2 = _apply_k2(degp, P2, W2, b2.reshape(1, -1))
    P3 = _seg4(g2, edges16)
    prob, h = _apply_k3(degp, P3, W3, b3.reshape(1, -1), W_out, b_out.reshape(1, -1))
    return (prob[:n], h[:n])
